# Initial kernel scaffold; baseline (speedup 1.0000x reference)
import jax
import jax.numpy as jnp
from jax.experimental import pallas as pl


def kernel(inv_glyphs, emb, W1, b1, gamma, beta, W2, b2):
    raise NotImplementedError("write your pallas kernel here")


# speedup vs baseline: 13.8101x; 13.8101x over previous
import functools

import jax
import jax.numpy as jnp
from jax import lax
from jax.experimental import pallas as pl
from jax.experimental.pallas import tpu as pltpu
from jax.experimental.pallas import tpu_sc as plsc

VOCAB = 5977
INV_SLOTS = 55
EDIM = 32
HDIM = 128
BATCH = 16384

NC = 2
NS = 16
NW = NC * NS

N_ROWS = BATCH * INV_SLOTS
ROWS_PER_W = N_ROWS // NW
CHUNK = 2816
N_CHUNKS = ROWS_PER_W // CHUNK


def _gather_body(idx_hbm, emb_hbm, out_hbm, idx_v, rows_v, sem):
    wid = lax.axis_index("s") * NC + lax.axis_index("c")
    base = wid * ROWS_PER_W
    for k in range(N_CHUNKS):
        off = base + k * CHUNK
        pltpu.sync_copy(idx_hbm.at[pl.ds(off, CHUNK)], idx_v)
        pltpu.async_copy(emb_hbm.at[idx_v], rows_v, sem).wait()
        pltpu.sync_copy(rows_v, out_hbm.at[pl.ds(off, CHUNK)])


@functools.cache
def _sc_gather():
    return pl.kernel(
        _gather_body,
        out_type=jax.ShapeDtypeStruct((N_ROWS, EDIM), jnp.float32),
        mesh=plsc.VectorSubcoreMesh(core_axis_name="c", subcore_axis_name="s"),
        scratch_types=[
            pltpu.VMEM((CHUNK,), jnp.int32),
            pltpu.VMEM((CHUNK, EDIM), jnp.float32),
            pltpu.SemaphoreType.DMA,
        ],
        compiler_params=pltpu.CompilerParams(use_tc_tiling_on_sc=False),
    )


def _mlp_body(x_ref, w1_ref, b1_ref, g_ref, bt_ref, w2_ref, b2_ref, o_ref):
    x = x_ref[...]
    h = jnp.dot(x, w1_ref[...], preferred_element_type=jnp.float32) + b1_ref[...]
    mean = jnp.mean(h, axis=1, keepdims=True)
    var = jnp.mean((h - mean) ** 2, axis=1, keepdims=True)
    ln = (h - mean) * lax.rsqrt(var + 1e-5) * g_ref[...] + bt_ref[...]
    a = jnp.where(ln > 0, ln, jnp.exp(ln) - 1.0)
    o_ref[...] = jnp.dot(a, w2_ref[...], preferred_element_type=jnp.float32) + b2_ref[...]


def _mlp(x, W1, b1, gamma, beta, W2, b2, block_b=1024):
    in_dim = x.shape[1]
    grid = (x.shape[0] // block_b,)
    return pl.pallas_call(
        _mlp_body,
        grid=grid,
        in_specs=[
            pl.BlockSpec((block_b, in_dim), lambda i: (i, 0)),
            pl.BlockSpec((in_dim, HDIM), lambda i: (0, 0)),
            pl.BlockSpec((1, HDIM), lambda i: (0, 0)),
            pl.BlockSpec((1, HDIM), lambda i: (0, 0)),
            pl.BlockSpec((1, HDIM), lambda i: (0, 0)),
            pl.BlockSpec((HDIM, HDIM), lambda i: (0, 0)),
            pl.BlockSpec((1, HDIM), lambda i: (0, 0)),
        ],
        out_specs=pl.BlockSpec((block_b, HDIM), lambda i: (i, 0)),
        out_shape=jax.ShapeDtypeStruct((x.shape[0], HDIM), jnp.float32),
        compiler_params=pltpu.CompilerParams(
            dimension_semantics=("arbitrary",),
        ),
    )(x, W1, b1, gamma, beta, W2, b2)


def kernel(inv_glyphs, emb, W1, b1, gamma, beta, W2, b2):
    idx = inv_glyphs.reshape(-1).astype(jnp.int32)
    rows = _sc_gather()(idx, emb)
    x = rows.reshape(BATCH, INV_SLOTS * EDIM)
    return _mlp(x, W1, b1.reshape(1, HDIM), gamma.reshape(1, HDIM),
                beta.reshape(1, HDIM), W2, b2.reshape(1, HDIM))
